# baseline (speedup 1.0000x reference)
import functools

import jax
import jax.numpy as jnp
from jax import lax
from jax.experimental import pallas as pl
from jax.experimental.pallas import tpu as pltpu
from jax.experimental.pallas import tpu_sc as plsc

NC, NS, L = 2, 16, 16
NW = NC * NS
B = 16384
N = 50
NPAD = 64
D = 32
BPW = B // NW
CHUNK = 32
NCHUNKS = BPW // CHUNK
ROWS = CHUNK * N


def _sc_dists(table_hbm, i_hbm, ks_hbm, out_hbm,
              iidx_v, anch_v, ksidx_v, cand_v, dist_v, sem):
    wid = lax.axis_index("s") * NC + lax.axis_index("c")
    abase = wid * BPW

    pltpu.sync_copy(i_hbm.at[pl.ds(abase, BPW)], iidx_v)
    pltpu.async_copy(table_hbm.at[iidx_v], anch_v, sem).wait()

    zero16 = jnp.zeros((L,), jnp.float32)
    for r in range(ROWS, ROWS + L):
        cand_v[r, 0:L] = zero16
        cand_v[r, L:D] = zero16

    lane = lax.iota(jnp.int32, L)
    sgn0 = jnp.where(lane == 0, 1.0, -1.0)

    def chunk_body(c, _):
        cbase = abase + c * CHUNK
        pltpu.sync_copy(ks_hbm.at[pl.ds(cbase, CHUNK), :], ksidx_v)
        copies = [pltpu.async_copy(table_hbm.at[ksidx_v.at[al]],
                                   cand_v.at[pl.ds(al * N, N)], sem)
                  for al in range(CHUNK)]
        for cp in copies:
            cp.wait()

        def anchor_body(al, _):
            u_lo = anch_v[c * CHUNK + al, 0:L]
            u_hi = anch_v[c * CHUNK + al, L:D]
            c_lo = u_lo * sgn0
            c_hi = -u_hi
            r0 = al * N
            for g in range(4):
                rows = r0 + g * L + lane
                acc = jnp.zeros((L,), jnp.float32)
                for j in range(D):
                    cj = c_lo[j] if j < L else c_hi[j - L]
                    kj = plsc.load_gather(
                        cand_v, [rows, jnp.full((L,), j, jnp.int32)])
                    acc = acc + cj * kj
                dist_v[al, pl.ds(g * L, L)] = acc
            return _

        lax.fori_loop(0, CHUNK, anchor_body, None)
        pltpu.sync_copy(dist_v, out_hbm.at[pl.ds(cbase, CHUNK)])
        return _

    lax.fori_loop(0, NCHUNKS, chunk_body, None)


_sc_kernel = functools.partial(
    pl.kernel,
    out_type=jax.ShapeDtypeStruct((B, NPAD), jnp.float32),
    mesh=plsc.VectorSubcoreMesh(core_axis_name="c", subcore_axis_name="s",
                                num_cores=NC, num_subcores=NS),
    compiler_params=pltpu.CompilerParams(needs_layout_passes=False,
                                         use_tc_tiling_on_sc=False),
    scratch_types=[
        pltpu.VMEM((BPW,), jnp.int32),
        pltpu.VMEM((BPW, D), jnp.float32),
        pltpu.VMEM((CHUNK, N), jnp.int32),
        pltpu.VMEM((ROWS + L, D), jnp.float32),
        pltpu.VMEM((CHUNK, NPAD), jnp.float32),
        pltpu.SemaphoreType.DMA,
    ],
)(_sc_dists)


TC_BLK = 1024


def _tc_loss(d_ref, o_ref):
    d = d_ref[...]
    col = lax.broadcasted_iota(jnp.int32, d.shape, 1)
    d = jnp.where(d <= 1.0, jnp.float32(1.0 + 1e-06), d)
    a = -jnp.log(d + jnp.sqrt(d * d - 1.0))
    e = jnp.where(col < N, jnp.exp(a), 0.0)
    o_ref[...] = jnp.log(jnp.sum(e, axis=1) + 1e-06) - a[:, 0]


def kernel(table, I, Ks):
    dists = _sc_kernel(table, I.astype(jnp.int32), Ks.astype(jnp.int32))
    return pl.pallas_call(
        _tc_loss,
        grid=(B // TC_BLK,),
        in_specs=[pl.BlockSpec((TC_BLK, NPAD), lambda i: (i, 0))],
        out_specs=pl.BlockSpec((TC_BLK,), lambda i: (i,)),
        out_shape=jax.ShapeDtypeStruct((B,), jnp.float32),
    )(dists)


# speedup vs baseline: 1.4599x; 1.4599x over previous
import functools

import jax
import jax.numpy as jnp
from jax import lax
from jax.experimental import pallas as pl
from jax.experimental.pallas import tpu as pltpu
from jax.experimental.pallas import tpu_sc as plsc

NC, NS, L = 2, 16, 16
NW = NC * NS
B = 16384
N = 50
NPAD = 64
D = 32
BPW = B // NW
CHUNK = 32
NCHUNKS = BPW // CHUNK
ROWS = CHUNK * N


def _sc_dists(table_hbm, i_hbm, ks_hbm, out_hbm,
              iidx_v, anch_v, ksidx_v, cand_v, dist_v, coef_v, sem):
    wid = lax.axis_index("s") * NC + lax.axis_index("c")
    abase = wid * BPW

    pltpu.sync_copy(i_hbm.at[pl.ds(abase, BPW)], iidx_v)
    pltpu.async_copy(table_hbm.at[iidx_v], anch_v, sem).wait()

    zero16 = jnp.zeros((L,), jnp.float32)
    for r in range(ROWS, ROWS + L):
        cand_v[r, 0:L] = zero16
        cand_v[r, L:D] = zero16

    lane = lax.iota(jnp.int32, L)
    sgn0 = jnp.where(lane == 0, 1.0, -1.0)

    def chunk_body(c, _):
        cbase = abase + c * CHUNK
        pltpu.sync_copy(ks_hbm.at[pl.ds(cbase, CHUNK), :], ksidx_v)
        copies = [pltpu.async_copy(table_hbm.at[ksidx_v.at[al]],
                                   cand_v.at[pl.ds(al * N, N)], sem)
                  for al in range(CHUNK)]
        for cp in copies:
            cp.wait()

        def anchor_body(al, _):
            u_lo = anch_v[c * CHUNK + al, 0:L]
            u_hi = anch_v[c * CHUNK + al, L:D]
            coef_v[0:L] = u_lo * sgn0
            coef_v[L:D] = -u_hi
            cols = [(j + lane) & (D - 1) for j in range(D)]
            cjs = [plsc.load_gather(coef_v, [cols[j]]) for j in range(D)]
            r0 = al * N
            for g in range(4):
                rows = r0 + g * L + lane
                acc = jnp.zeros((L,), jnp.float32)
                for j in range(D):
                    kj = plsc.load_gather(cand_v, [rows, cols[j]])
                    acc = acc + cjs[j] * kj
                dist_v[al, pl.ds(g * L, L)] = acc
            return _

        lax.fori_loop(0, CHUNK, anchor_body, None)
        pltpu.sync_copy(dist_v, out_hbm.at[pl.ds(cbase, CHUNK)])
        return _

    lax.fori_loop(0, NCHUNKS, chunk_body, None)


_sc_kernel = functools.partial(
    pl.kernel,
    out_type=jax.ShapeDtypeStruct((B, NPAD), jnp.float32),
    mesh=plsc.VectorSubcoreMesh(core_axis_name="c", subcore_axis_name="s",
                                num_cores=NC, num_subcores=NS),
    compiler_params=pltpu.CompilerParams(needs_layout_passes=False,
                                         use_tc_tiling_on_sc=False),
    scratch_types=[
        pltpu.VMEM((BPW,), jnp.int32),
        pltpu.VMEM((BPW, D), jnp.float32),
        pltpu.VMEM((CHUNK, N), jnp.int32),
        pltpu.VMEM((ROWS + L, D), jnp.float32),
        pltpu.VMEM((CHUNK, NPAD), jnp.float32),
        pltpu.VMEM((D,), jnp.float32),
        pltpu.SemaphoreType.DMA,
    ],
)(_sc_dists)


TC_BLK = 1024


def _tc_loss(d_ref, o_ref):
    d = d_ref[...]
    col = lax.broadcasted_iota(jnp.int32, d.shape, 1)
    d = jnp.where(d <= 1.0, jnp.float32(1.0 + 1e-06), d)
    a = -jnp.log(d + jnp.sqrt(d * d - 1.0))
    e = jnp.where(col < N, jnp.exp(a), 0.0)
    o_ref[...] = jnp.log(jnp.sum(e, axis=1) + 1e-06) - a[:, 0]


def kernel(table, I, Ks):
    dists = _sc_kernel(table, I.astype(jnp.int32), Ks.astype(jnp.int32))
    return pl.pallas_call(
        _tc_loss,
        grid=(B // TC_BLK,),
        in_specs=[pl.BlockSpec((TC_BLK, NPAD), lambda i: (i, 0))],
        out_specs=pl.BlockSpec((TC_BLK,), lambda i: (i,)),
        out_shape=jax.ShapeDtypeStruct((B,), jnp.float32),
    )(dists)
